# baseline (speedup 1.0000x reference)
import functools

import jax
import jax.numpy as jnp
from jax import lax
from jax.experimental import pallas as pl
from jax.experimental.pallas import tpu as pltpu
from jax.experimental.pallas import tpu_sc as plsc

N = 524288
C = 32
CH = 32768
NCHUNK = 2 * C * N // CH
CPT = NCHUNK // 32
R = 4096
L = 128

_mesh = plsc.VectorSubcoreMesh(core_axis_name="c", subcore_axis_name="s")


@functools.partial(
    pl.kernel,
    out_type=jax.ShapeDtypeStruct((NCHUNK, CH), jnp.int32),
    mesh=_mesh,
    compiler_params=pltpu.CompilerParams(needs_layout_passes=False),
    scratch_types=[
        pltpu.VMEM((CH,), jnp.int32),
        pltpu.VMEM((CH,), jnp.int32),
        pltpu.VMEM((2048,), jnp.int32),
        pltpu.VMEM((CH,), jnp.int32),
    ],
)
def _sc_chunk_sort(in_hbm, out_hbm, a_v, b_v, hist_v, hist32_v):
    cid = lax.axis_index("c")
    sid = lax.axis_index("s")
    wid = sid * 2 + cid
    iota = lax.iota(jnp.int32, 16)
    minint = jnp.int32(-2147483648)

    def digits_of(src, l, shift, mask, descm):
        k = src[pl.ds(l * 16, 16)]
        sgn = lax.shift_right_logical(k, jnp.full((16,), 31, jnp.int32))
        ks = (k ^ ((0 - sgn) | minint)) ^ descm
        dig = lax.shift_right_logical(ks, jnp.full((16,), shift, jnp.int32)) & mask
        return k, dig

    def rankpipe(dig):
        sk, sv = plsc.sort_key_val(dig * 16 + iota, iota)
        sd = lax.shift_right_logical(sk, jnp.full((16,), 4, jnp.int32))
        prev = sd[jnp.maximum(iota - 1, 0)]
        st = jnp.logical_or(iota == 0, sd != prev)
        start = plsc.cummax(jnp.where(st, iota, 0))
        occ = iota - start
        nxt = sd[jnp.minimum(iota + 1, 15)]
        lastm = jnp.logical_or(iota == 15, sd != nxt)
        return sd, sv, occ, lastm

    ones16 = jnp.full((16,), 1, jnp.int32)
    zeros16 = jnp.zeros((16,), jnp.int32)

    def do_pass(src, dst, shift, mask, descm):
        @plsc.parallel_loop(0, CH // 16, unroll=4)
        def hist(l):
            _, dig = digits_of(src, l, shift, mask, descm)
            plsc.addupdate_scatter(hist32_v, [iota * 2048 + dig], ones16)

        def scan(i, carry):
            acc = zeros16
            for l in range(16):
                v = hist32_v[pl.ds(l * 2048 + i * 16, 16)]
                hist32_v[pl.ds(l * 2048 + i * 16, 16)] = zeros16
                acc = acc + v
            s = plsc.cumsum(acc)
            hist_v[pl.ds(i * 16, 16)] = s - acc + carry
            return carry + jnp.sum(acc)

        lax.fori_loop(0, 128, scan, 0)

        def perm(l2, c2):
            for u in range(4):
                l = l2 * 4 + u
                k, dig = digits_of(src, l, shift, mask, descm)
                sd, sv, occ, lastm = rankpipe(dig)
                base = plsc.load_gather(hist_v, [sd])
                dest = base + occ
                plsc.store_scatter(hist_v, [sd], dest + 1, mask=lastm)
                ksort = k[sv]
                plsc.store_scatter(dst, [dest], ksort)
            return c2

        lax.fori_loop(0, CH // 64, perm, 0)

    @plsc.parallel_loop(0, CH // 16, unroll=4)
    def zero32(i):
        hist32_v[pl.ds(i * 16, 16)] = jnp.zeros((16,), jnp.int32)

    def chunk_body(t, carry):
        m = t * 32 + wid
        descm = jnp.full((16,), 0, jnp.int32) - (m & 1)
        pltpu.sync_copy(in_hbm.at[m], a_v)
        do_pass(a_v, b_v, 0, 2047, descm)
        do_pass(b_v, a_v, 11, 2047, descm)
        do_pass(a_v, b_v, 22, 1023, descm)
        pltpu.sync_copy(b_v, out_hbm.at[m])
        return carry

    lax.fori_loop(0, CPT, chunk_body, 0)


BLK = 64
NBLK = R // BLK
NPAIR = R // (2 * BLK)


def merge_body_factory(kmin, kmax, n, c):
    def body(x_ref, g_ref, out_ref, a_ref, b_ref):
        col = pl.program_id(0)

        @pl.when(col == 0)
        def _():
            out_ref[0, 0] = 0.0

        a_ref[...] = x_ref[0]
        b_ref[...] = g_ref[0]

        c_iota = lax.broadcasted_iota(jnp.int32, (BLK, L), 1)
        r_iota = lax.broadcasted_iota(jnp.int32, (BLK, L), 0)

        for k in range(kmin, kmax + 1):
            lm = (1 << k) >> 12
            desc = (c_iota & lm) != 0

            for ref in (a_ref, b_ref):
                for jexp in range(k - 1, 11, -1):
                    dl = 1 << (jexp - 12)
                    bit = (c_iota & dl) != 0
                    want_min = bit == desc

                    def lane_blk(t, _, ref=ref, dl=dl, bit=bit, wm=want_min):
                        v = ref[pl.ds(t * BLK, BLK)]
                        up = pltpu.roll(v, L - dl, 1)
                        dn = pltpu.roll(v, dl, 1)
                        p = jnp.where(bit, dn, up)
                        ref[pl.ds(t * BLK, BLK)] = jnp.where(
                            wm, jnp.minimum(v, p), jnp.maximum(v, p))
                        return 0

                    lax.fori_loop(0, NBLK, lane_blk, 0)

                def row_pass(s, _, ref=ref, desc=desc):
                    jexp = 11 - s

                    def pair_blk(t, __):
                        sh = jexp - 6
                        q = lax.shift_right_logical(t, sh)
                        rem = t & (lax.shift_left(1, sh) - 1)
                        lo = lax.shift_left(q, jexp + 1) + lax.shift_left(rem, 6)
                        dr = lax.shift_left(1, jexp)
                        vlo = ref[pl.ds(lo, BLK)]
                        vhi = ref[pl.ds(lo + dr, BLK)]
                        mn = jnp.minimum(vlo, vhi)
                        mx = jnp.maximum(vlo, vhi)
                        ref[pl.ds(lo, BLK)] = jnp.where(desc, mx, mn)
                        ref[pl.ds(lo + dr, BLK)] = jnp.where(desc, mn, mx)
                        return 0

                    lax.fori_loop(0, NPAIR, pair_blk, 0)
                    return 0

                lax.fori_loop(0, 6, row_pass, 0)

                def fused_blk(t, _, ref=ref, desc=desc):
                    v = ref[pl.ds(t * BLK, BLK)]
                    for d in (32, 16, 8, 4, 2, 1):
                        bit = (r_iota & d) != 0
                        p = jnp.where(bit, pltpu.roll(v, d, 0),
                                      pltpu.roll(v, BLK - d, 0))
                        wm = bit == desc
                        v = jnp.where(wm, jnp.minimum(v, p), jnp.maximum(v, p))
                    ref[pl.ds(t * BLK, BLK)] = v
                    return 0

                lax.fori_loop(0, NBLK, fused_blk, 0)

        x = b_ref[...] - a_ref[...]
        loss = jnp.maximum(x, 0.0) - x + jnp.log1p(jnp.exp(-jnp.abs(x)))
        out_ref[0, 0] += jnp.sum(loss)

    return body


def kernel(true_data, fake_data):
    bits = jax.lax.bitcast_convert_type(
        jnp.concatenate(
            [true_data.T.reshape(-1), fake_data.T.reshape(-1)]
        ),
        jnp.int32,
    ).reshape(NCHUNK, CH)
    sorted_chunks = _sc_chunk_sort(bits)
    f = jax.lax.bitcast_convert_type(sorted_chunks, jnp.float32)
    f = f.reshape(2, C, 16, 8, R).transpose(0, 1, 4, 2, 3).reshape(2, C, R, L)
    tx = f[0]
    tg = f[1]

    total = pl.pallas_call(
        merge_body_factory(16, 19, N, C),
        grid=(C,),
        in_specs=[
            pl.BlockSpec((1, R, L), lambda col: (col, 0, 0)),
            pl.BlockSpec((1, R, L), lambda col: (col, 0, 0)),
        ],
        out_specs=pl.BlockSpec(memory_space=pltpu.SMEM),
        out_shape=jax.ShapeDtypeStruct((1, 1), jnp.float32),
        scratch_shapes=[
            pltpu.VMEM((R, L), jnp.float32),
            pltpu.VMEM((R, L), jnp.float32),
        ],
    )(tx, tg)
    return total[0, 0] / (N * C)


# speedup vs baseline: 1.0369x; 1.0369x over previous
import functools

import jax
import jax.numpy as jnp
from jax import lax
from jax.experimental import pallas as pl
from jax.experimental.pallas import tpu as pltpu
from jax.experimental.pallas import tpu_sc as plsc

N = 524288
C = 32
CH = 32768
NCHUNK = 2 * C * N // CH
CPT = NCHUNK // 32
R = 4096
L = 128

_mesh = plsc.VectorSubcoreMesh(core_axis_name="c", subcore_axis_name="s")


@functools.partial(
    pl.kernel,
    out_type=jax.ShapeDtypeStruct((NCHUNK, CH), jnp.int32),
    mesh=_mesh,
    compiler_params=pltpu.CompilerParams(needs_layout_passes=False),
    scratch_types=[
        pltpu.VMEM((CH,), jnp.int32),
        pltpu.VMEM((CH,), jnp.int32),
        pltpu.VMEM((2048,), jnp.int32),
        pltpu.VMEM((CH,), jnp.int32),
    ],
)
def _sc_chunk_sort(in_hbm, out_hbm, a_v, b_v, hist_v, hist32_v):
    cid = lax.axis_index("c")
    sid = lax.axis_index("s")
    wid = sid * 2 + cid
    iota = lax.iota(jnp.int32, 16)
    minint = jnp.int32(-2147483648)

    def digits_of(src, l, shift, mask, descm):
        k = src[pl.ds(l * 16, 16)]
        sgn = lax.shift_right_logical(k, jnp.full((16,), 31, jnp.int32))
        ks = (k ^ ((0 - sgn) | minint)) ^ descm
        dig = lax.shift_right_logical(ks, jnp.full((16,), shift, jnp.int32)) & mask
        return k, dig

    def rankpipe(dig):
        sk, sv = plsc.sort_key_val(dig * 16 + iota, iota)
        sd = lax.shift_right_logical(sk, jnp.full((16,), 4, jnp.int32))
        prev = sd[jnp.maximum(iota - 1, 0)]
        st = jnp.logical_or(iota == 0, sd != prev)
        start = plsc.cummax(jnp.where(st, iota, 0))
        occ = iota - start
        nxt = sd[jnp.minimum(iota + 1, 15)]
        lastm = jnp.logical_or(iota == 15, sd != nxt)
        return sd, sv, occ, lastm

    ones16 = jnp.full((16,), 1, jnp.int32)
    zeros16 = jnp.zeros((16,), jnp.int32)

    def do_pass(src, dst, shift, mask, descm):
        @plsc.parallel_loop(0, CH // 16, unroll=4)
        def hist(l):
            _, dig = digits_of(src, l, shift, mask, descm)
            plsc.addupdate_scatter(hist32_v, [iota * 2048 + dig], ones16)

        def scan(i, carry):
            acc = zeros16
            for l in range(16):
                v = hist32_v[pl.ds(l * 2048 + i * 16, 16)]
                hist32_v[pl.ds(l * 2048 + i * 16, 16)] = zeros16
                acc = acc + v
            s = plsc.cumsum(acc)
            hist_v[pl.ds(i * 16, 16)] = s - acc + carry
            return carry + jnp.sum(acc)

        lax.fori_loop(0, 128, scan, 0)

        def perm(l2, c2):
            for u in range(4):
                l = l2 * 4 + u
                k, dig = digits_of(src, l, shift, mask, descm)
                sd, sv, occ, lastm = rankpipe(dig)
                base = plsc.load_gather(hist_v, [sd])
                dest = base + occ
                plsc.store_scatter(hist_v, [sd], dest + 1, mask=lastm)
                ksort = k[sv]
                plsc.store_scatter(dst, [dest], ksort)
            return c2

        lax.fori_loop(0, CH // 64, perm, 0)

    @plsc.parallel_loop(0, CH // 16, unroll=4)
    def zero32(i):
        hist32_v[pl.ds(i * 16, 16)] = jnp.zeros((16,), jnp.int32)

    def chunk_body(t, carry):
        m = t * 32 + wid
        descm = jnp.full((16,), 0, jnp.int32) - (m & 1)
        pltpu.sync_copy(in_hbm.at[m], a_v)
        do_pass(a_v, b_v, 0, 2047, descm)
        do_pass(b_v, a_v, 11, 2047, descm)
        do_pass(a_v, b_v, 22, 1023, descm)
        pltpu.sync_copy(b_v, out_hbm.at[m])
        return carry

    lax.fori_loop(0, CPT, chunk_body, 0)


BLK = 64
NBLK = R // BLK
NPAIR = R // (2 * BLK)


def merge_body_factory(kmin, kmax, n, c):
    def body(x_ref, g_ref, out_ref, a_ref, b_ref):
        col = pl.program_id(0)

        @pl.when(col == 0)
        def _():
            out_ref[0, 0] = 0.0

        a_ref[...] = x_ref[0]
        b_ref[...] = g_ref[0]

        c_iota = lax.broadcasted_iota(jnp.int32, (BLK, L), 1)
        r_iota = lax.broadcasted_iota(jnp.int32, (BLK, L), 0)

        for k in range(kmin, kmax + 1):
            lm = (1 << k) >> 12
            desc = (c_iota & lm) != 0

            lane_jexps = tuple(range(k - 1, 11, -1))

            for ref in (a_ref, b_ref):
                def lane_blk(t, _, ref=ref):
                    v = ref[pl.ds(t * BLK, BLK)]
                    for jexp in lane_jexps:
                        dl = 1 << (jexp - 12)
                        bit = (c_iota & dl) != 0
                        wm = bit == desc
                        up = pltpu.roll(v, L - dl, 1)
                        dn = pltpu.roll(v, dl, 1)
                        p = jnp.where(bit, dn, up)
                        v = jnp.where(wm, jnp.minimum(v, p), jnp.maximum(v, p))
                    ref[pl.ds(t * BLK, BLK)] = v
                    return 0

                lax.fori_loop(0, NBLK, lane_blk, 0)

                def row_pass(s, _, ref=ref, desc=desc):
                    jexp = 11 - s

                    def pair_blk(t, __):
                        sh = jexp - 6
                        q = lax.shift_right_logical(t, sh)
                        rem = t & (lax.shift_left(1, sh) - 1)
                        lo = lax.shift_left(q, jexp + 1) + lax.shift_left(rem, 6)
                        dr = lax.shift_left(1, jexp)
                        vlo = ref[pl.ds(lo, BLK)]
                        vhi = ref[pl.ds(lo + dr, BLK)]
                        mn = jnp.minimum(vlo, vhi)
                        mx = jnp.maximum(vlo, vhi)
                        ref[pl.ds(lo, BLK)] = jnp.where(desc, mx, mn)
                        ref[pl.ds(lo + dr, BLK)] = jnp.where(desc, mn, mx)
                        return 0

                    lax.fori_loop(0, NPAIR, pair_blk, 0)
                    return 0

                lax.fori_loop(0, 6, row_pass, 0)

                def fused_blk(t, _, ref=ref, desc=desc):
                    v = ref[pl.ds(t * BLK, BLK)]
                    for d in (32, 16, 8, 4, 2, 1):
                        bit = (r_iota & d) != 0
                        p = jnp.where(bit, pltpu.roll(v, d, 0),
                                      pltpu.roll(v, BLK - d, 0))
                        wm = bit == desc
                        v = jnp.where(wm, jnp.minimum(v, p), jnp.maximum(v, p))
                    ref[pl.ds(t * BLK, BLK)] = v
                    return 0

                lax.fori_loop(0, NBLK, fused_blk, 0)

        x = b_ref[...] - a_ref[...]
        loss = jnp.maximum(x, 0.0) - x + jnp.log1p(jnp.exp(-jnp.abs(x)))
        out_ref[0, 0] += jnp.sum(loss)

    return body


def kernel(true_data, fake_data):
    bits = jax.lax.bitcast_convert_type(
        jnp.concatenate(
            [true_data.T.reshape(-1), fake_data.T.reshape(-1)]
        ),
        jnp.int32,
    ).reshape(NCHUNK, CH)
    sorted_chunks = _sc_chunk_sort(bits)
    f = jax.lax.bitcast_convert_type(sorted_chunks, jnp.float32)
    f = f.reshape(2, C, 16, 8, R).transpose(0, 1, 4, 2, 3).reshape(2, C, R, L)
    tx = f[0]
    tg = f[1]

    total = pl.pallas_call(
        merge_body_factory(16, 19, N, C),
        grid=(C,),
        in_specs=[
            pl.BlockSpec((1, R, L), lambda col: (col, 0, 0)),
            pl.BlockSpec((1, R, L), lambda col: (col, 0, 0)),
        ],
        out_specs=pl.BlockSpec(memory_space=pltpu.SMEM),
        out_shape=jax.ShapeDtypeStruct((1, 1), jnp.float32),
        scratch_shapes=[
            pltpu.VMEM((R, L), jnp.float32),
            pltpu.VMEM((R, L), jnp.float32),
        ],
    )(tx, tg)
    return total[0, 0] / (N * C)
